# baseline (speedup 1.0000x reference)
import functools

import jax
import jax.numpy as jnp
from jax import lax
from jax.experimental import pallas as pl
from jax.experimental.pallas import tpu as pltpu
from jax.experimental.pallas import tpu_sc as plsc

N = 10000
NP = 10240
D = 256
HD = 128
G4 = 4 * 256
E = 160000

NC = 2
NS = 16
LANES = 16

CHUNK = 80
EPT = 10240
NCHUNKS = EPT // CHUNK
E_PAD = NS * EPT
ROWS_PER_TILE = NP // NS

BLK = 1000


NBUF = 4
DRING = 8


def _sc_agg_body(x_hbm, aux_hbm, didx_hbm, out_hbm, acc_sh,
                 a0, a1, a2, a3, rb0, rb1, rb2, rb3,
                 d0, d1, d2, d3, d4, d5, d6, d7,
                 ga0, ga1, ga2, ga3, gg0, gg1, gg2, gg3,
                 gs0, gs1, gs2, gs3, gd0, gd1, gd2, gd3, gd5, gd6, gd7, gd8):
    c = lax.axis_index("c")
    sid = lax.axis_index("s")
    row0 = sid * ROWS_PER_TILE
    zv = jnp.zeros((LANES,), jnp.float32)

    def zrow(e, carry):
        for j in range(HD // LANES):
            rb0[e, pl.ds(j * LANES, LANES)] = zv
        return carry

    lax.fori_loop(0, CHUNK, zrow, 0, unroll=False)
    for i in range(ROWS_PER_TILE // CHUNK):
        pltpu.sync_copy(rb0, acc_sh.at[pl.ds(row0 + i * CHUNK, CHUNK)])
    plsc.subcore_barrier()

    aux = [a0, a1, a2, a3]
    rows = [rb0, rb1, rb2, rb3]
    didx = [d0, d1, d2, d3, d4, d5, d6, d7]
    asem = [ga0, ga1, ga2, ga3]
    gsem = [gg0, gg1, gg2, gg3]
    ssem = [gs0, gs1, gs2, gs3]
    dsem = [gd0, gd1, gd2, gd3, gd5, gd6, gd7, gd8]

    def issue_aux(ch, b):
        pltpu.async_copy(aux_hbm.at[c, sid, ch], aux[b], asem[b])

    def wait_aux(b):
        pltpu.make_async_copy(aux_hbm.at[c, sid, 0], aux[b], asem[b]).wait()

    def issue_didx(ch, b):
        pltpu.async_copy(didx_hbm.at[sid, ch], didx[b], dsem[b])

    def wait_didx(b):
        pltpu.make_async_copy(didx_hbm.at[sid, 0], didx[b], dsem[b]).wait()

    def issue_gather(b):
        pltpu.async_copy(x_hbm.at[aux[b].at[pl.ds(0, CHUNK)]], rows[b], gsem[b])

    def wait_gather(b):
        pltpu.make_async_copy(x_hbm.at[aux[0].at[pl.ds(0, CHUNK)]], rows[b],
                              gsem[b]).wait()

    def issue_scatter(b, db):
        pltpu.async_copy(rows[b], acc_sh.at[didx[db]], ssem[b], add=True)

    def wait_scatter(b):
        pltpu.make_async_copy(rows[b], acc_sh.at[didx[0]], ssem[b]).wait()

    def scale(b):
        @plsc.parallel_loop(0, CHUNK // LANES, 1)
        def grp(g):
            wg = plsc.bitcast(aux[b][pl.ds(CHUNK + g * LANES, LANES)],
                              jnp.float32)
            for el in range(LANES):
                spl = jnp.take_along_axis(wg, jnp.full((LANES,), el, jnp.int32),
                                          axis=0)
                e = g * LANES + el
                for j in range(HD // LANES):
                    sl = pl.ds(j * LANES, LANES)
                    rows[b][e, sl] = rows[b][e, sl] * spl

    for j in range(NBUF):
        issue_aux(j, j)
    for j in range(6):
        issue_didx(j, j)
    for j in range(2):
        wait_aux(j)
        issue_gather(j)

    def outer(k, carry):
        for b8 in range(DRING):
            ch = k * DRING + b8
            b = b8 % NBUF
            wait_gather(b)
            scale(b)
            wait_didx(b8)
            issue_scatter(b, b8)
            nb = (b + 2) % NBUF

            @pl.when(ch + 2 < NCHUNKS)
            def _():
                @pl.when(ch >= 2)
                def _():
                    wait_scatter(nb)
                wait_aux(nb)
                issue_gather(nb)

            @pl.when(ch + NBUF < NCHUNKS)
            def _():
                issue_aux(ch + NBUF, b)

            @pl.when(ch + 6 < NCHUNKS)
            def _():
                issue_didx(ch + 6, (b8 + 6) % DRING)
        return carry

    lax.fori_loop(0, NCHUNKS // DRING, outer, 0, unroll=False)
    for j in range(NBUF):
        wait_scatter(j)
    plsc.subcore_barrier()
    pltpu.sync_copy(acc_sh.at[pl.ds(row0, ROWS_PER_TILE)],
                    out_hbm.at[pl.ds(c * NP + row0, ROWS_PER_TILE)])


_sc_agg = functools.partial(
    pl.kernel,
    mesh=plsc.VectorSubcoreMesh(core_axis_name="c", subcore_axis_name="s"),
    compiler_params=pltpu.CompilerParams(needs_layout_passes=False),
    out_type=jax.ShapeDtypeStruct((2 * NP, HD), jnp.float32),
    scratch_types=(
        [pltpu.VMEM_SHARED((NP, HD), jnp.float32)]
        + [pltpu.VMEM((2 * CHUNK,), jnp.int32) for _ in range(NBUF)]
        + [pltpu.VMEM((CHUNK, HD), jnp.float32) for _ in range(NBUF)]
        + [pltpu.VMEM((CHUNK,), jnp.int32) for _ in range(DRING)]
        + [pltpu.SemaphoreType.DMA for _ in range(2 * NBUF + NBUF + DRING)]
    ),
)(_sc_agg_body)


def _gates(z):
    i = jax.nn.sigmoid(z[:, 0 * 256:1 * 256])
    g = jnp.tanh(z[:, 2 * 256:3 * 256])
    o = jax.nn.sigmoid(z[:, 3 * 256:4 * 256])
    cc = i * g
    hh = o * jnp.tanh(cc)
    return hh, cc


def _tc0_body(m_ref, x_ref, w_ref, b_ref, h1n_ref, c1n_ref):
    ma = x_ref[:, :HD] + m_ref[0]
    mb = x_ref[:, HD:] + m_ref[1]
    z = (jnp.dot(ma, w_ref[0], preferred_element_type=jnp.float32)
         + jnp.dot(mb, w_ref[1], preferred_element_type=jnp.float32)
         + b_ref[0:1, :])
    h1, c1 = _gates(z)
    h1n_ref[0] = h1
    c1n_ref[0] = c1


def _ln(x, g, b):
    m = jnp.mean(x, axis=-1, keepdims=True)
    v = jnp.mean((x - m) ** 2, axis=-1, keepdims=True)
    return (x - m) * jax.lax.rsqrt(v + 1e-5) * g + b


def _tc1_body(m_ref, w_ref, b_ref, g_ref, be_ref, ha_ref, ca_ref,
              h2_ref, c2_ref):
    hp = ha_ref[0]
    ma = hp[:, :HD] + m_ref[0]
    mb = hp[:, HD:] + m_ref[1]
    z = (jnp.dot(ma, w_ref[0], preferred_element_type=jnp.float32)
         + jnp.dot(mb, w_ref[1], preferred_element_type=jnp.float32)
         + b_ref[0:1, :])
    h2, c2 = _gates(z)
    h2_ref[0] = _ln(h2, g_ref[0:1, :], be_ref[0:1, :])
    c2_ref[0] = _ln(c2, g_ref[0:1, :], be_ref[0:1, :])


def _tc_cell0(m_split, x, w_split, b8):
    return pl.pallas_call(
        _tc0_body,
        grid=(N // BLK,),
        in_specs=[
            pl.BlockSpec((2, BLK, HD), lambda i: (0, i, 0)),
            pl.BlockSpec((BLK, D), lambda i: (i, 0)),
            pl.BlockSpec((2, HD, G4), lambda i: (0, 0, 0)),
            pl.BlockSpec((8, G4), lambda i: (0, 0)),
        ],
        out_specs=[
            pl.BlockSpec((1, BLK, D), lambda i: (0, i, 0)),
            pl.BlockSpec((1, BLK, D), lambda i: (0, i, 0)),
        ],
        out_shape=[
            jax.ShapeDtypeStruct((2, N, D), jnp.float32),
            jax.ShapeDtypeStruct((2, N, D), jnp.float32),
        ],
    )(m_split, x, w_split, b8)


def _tc_cell1(m_split, w_split, b8, g8, be8, h_all, c_all):
    return pl.pallas_call(
        _tc1_body,
        grid=(N // BLK,),
        in_specs=[
            pl.BlockSpec((2, BLK, HD), lambda i: (0, i, 0)),
            pl.BlockSpec((2, HD, G4), lambda i: (0, 0, 0)),
            pl.BlockSpec((8, G4), lambda i: (0, 0)),
            pl.BlockSpec((8, D), lambda i: (0, 0)),
            pl.BlockSpec((8, D), lambda i: (0, 0)),
            pl.BlockSpec((1, BLK, D), lambda i: (0, i, 0)),
            pl.BlockSpec(memory_space=pl.ANY),
        ],
        out_specs=[
            pl.BlockSpec((1, BLK, D), lambda i: (1, i, 0)),
            pl.BlockSpec((1, BLK, D), lambda i: (1, i, 0)),
        ],
        out_shape=[
            jax.ShapeDtypeStruct((2, N, D), jnp.float32),
            jax.ShapeDtypeStruct((2, N, D), jnp.float32),
        ],
        input_output_aliases={5: 0, 6: 1},
    )(m_split, w_split, b8, g8, be8, h_all, c_all)


def kernel(X, edge_index, edge_weight, Wx0, Wh0, b0, Wx1, Wh1, b1, ln_gamma, ln_beta):
    src = edge_index[0]
    dst = edge_index[1]
    pad = E_PAD - E
    pad_idx = jnp.arange(pad, dtype=jnp.int32) % N
    srcp = jnp.concatenate([src, pad_idx])
    dstp = jnp.concatenate([dst, pad_idx]).reshape(NS, NCHUNKS, CHUNK)
    wp = jnp.concatenate([edge_weight, jnp.zeros((pad,), jnp.float32)])
    wbits = jax.lax.bitcast_convert_type(wp, jnp.int32).reshape(NS, NCHUNKS, CHUNK)
    src2 = jnp.stack([2 * srcp, 2 * srcp + 1]).reshape(2, NS, NCHUNKS, CHUNK)
    auxp = jnp.concatenate(
        [src2, jnp.broadcast_to(wbits[None], (2, NS, NCHUNKS, CHUNK))], axis=-1)
    b0_8 = jnp.tile(b0[None, :], (8, 1))
    b1_8 = jnp.tile(b1[None, :], (8, 1))
    g8 = jnp.tile(ln_gamma[None, :], (8, 1))
    be8 = jnp.tile(ln_beta[None, :], (8, 1))
    w0s = Wx0.reshape(2, HD, G4)
    w1s = Wx1.reshape(2, HD, G4)

    m0 = _sc_agg(X.reshape(2 * N, HD), auxp, dstp)
    h_all, c_all = _tc_cell0(m0.reshape(2, NP, HD), X, w0s, b0_8)
    m1 = _sc_agg(h_all.reshape(4 * N, HD), auxp, dstp)
    hidden, cell = _tc_cell1(m1.reshape(2, NP, HD), w1s, b1_8, g8, be8,
                             h_all, c_all)
    return hidden, cell


# speedup vs baseline: 1.2998x; 1.2998x over previous
import functools

import jax
import jax.numpy as jnp
from jax import lax
from jax.experimental import pallas as pl
from jax.experimental.pallas import tpu as pltpu
from jax.experimental.pallas import tpu_sc as plsc

N = 10000
NP = 10240
D = 256
HD = 128
G4 = 4 * 256
E = 160000

NC = 2
NS = 16
LANES = 16

CHUNK = 80
EPT = 10240
NCHUNKS = EPT // CHUNK
E_PAD = NS * EPT
ROWS_PER_TILE = NP // NS

BLK = 1000


NBUF = 4
DRING = 8


def _sc_agg_body(x_hbm, aux_hbm, didx_hbm, out_hbm, acc_sh,
                 a0, a1, a2, a3, rb0, rb1, rb2, rb3,
                 d0, d1, d2, d3, d4, d5, d6, d7,
                 ga0, ga1, ga2, ga3, gg0, gg1, gg2, gg3,
                 gs0, gs1, gs2, gs3, gd0, gd1, gd2, gd3, gd5, gd6, gd7, gd8):
    c = lax.axis_index("c")
    sid = lax.axis_index("s")
    row0 = sid * ROWS_PER_TILE
    pltpu.sync_copy(x_hbm.at[pl.ds(c * NP + row0, ROWS_PER_TILE)],
                    acc_sh.at[pl.ds(row0, ROWS_PER_TILE)])
    plsc.subcore_barrier()

    aux = [a0, a1, a2, a3]
    rows = [rb0, rb1, rb2, rb3]
    didx = [d0, d1, d2, d3, d4, d5, d6, d7]
    asem = [ga0, ga1, ga2, ga3]
    gsem = [gg0, gg1, gg2, gg3]
    ssem = [gs0, gs1, gs2, gs3]
    dsem = [gd0, gd1, gd2, gd3, gd5, gd6, gd7, gd8]

    def issue_aux(ch, b):
        pltpu.async_copy(aux_hbm.at[c, sid, ch], aux[b], asem[b])

    def wait_aux(b):
        pltpu.make_async_copy(aux_hbm.at[c, sid, 0], aux[b], asem[b]).wait()

    def issue_didx(ch, b):
        pltpu.async_copy(didx_hbm.at[sid, ch], didx[b], dsem[b])

    def wait_didx(b):
        pltpu.make_async_copy(didx_hbm.at[sid, 0], didx[b], dsem[b]).wait()

    def issue_gather(b):
        pltpu.async_copy(x_hbm.at[aux[b].at[pl.ds(0, CHUNK)]], rows[b], gsem[b])

    def wait_gather(b):
        pltpu.make_async_copy(x_hbm.at[aux[0].at[pl.ds(0, CHUNK)]], rows[b],
                              gsem[b]).wait()

    def issue_scatter(b, db):
        pltpu.async_copy(rows[b], acc_sh.at[didx[db]], ssem[b], add=True)

    def wait_scatter(b):
        pltpu.make_async_copy(rows[b], acc_sh.at[didx[0]], ssem[b]).wait()

    def scale(b):
        def grp(g, carry):
            wg = plsc.bitcast(aux[b][pl.ds(CHUNK + g * LANES, LANES)],
                              jnp.float32)
            for el in range(LANES):
                spl = jnp.take_along_axis(wg, jnp.full((LANES,), el, jnp.int32),
                                          axis=0)
                e = g * LANES + el
                for j in range(HD // LANES):
                    sl = pl.ds(j * LANES, LANES)
                    rows[b][e, sl] = rows[b][e, sl] * spl
            return carry
        lax.fori_loop(0, CHUNK // LANES, grp, 0, unroll=False)

    for j in range(NBUF):
        issue_aux(j, j)
    for j in range(6):
        issue_didx(j, j)
    for j in range(2):
        wait_aux(j)
        issue_gather(j)

    def outer(k, carry):
        for b8 in range(DRING):
            ch = k * DRING + b8
            b = b8 % NBUF
            wait_gather(b)
            scale(b)
            wait_didx(b8)
            issue_scatter(b, b8)
            nb = (b + 2) % NBUF

            @pl.when(ch + 2 < NCHUNKS)
            def _():
                @pl.when(ch >= 2)
                def _():
                    wait_scatter(nb)
                wait_aux(nb)
                issue_gather(nb)

            @pl.when(ch + NBUF < NCHUNKS)
            def _():
                issue_aux(ch + NBUF, b)

            @pl.when(ch + 6 < NCHUNKS)
            def _():
                issue_didx(ch + 6, (b8 + 6) % DRING)
        return carry

    lax.fori_loop(0, NCHUNKS // DRING, outer, 0, unroll=False)
    for j in range(NBUF):
        wait_scatter(j)
    plsc.subcore_barrier()
    pltpu.sync_copy(acc_sh.at[pl.ds(row0, ROWS_PER_TILE)],
                    out_hbm.at[pl.ds(c * NP + row0, ROWS_PER_TILE)])


_sc_agg = functools.partial(
    pl.kernel,
    mesh=plsc.VectorSubcoreMesh(core_axis_name="c", subcore_axis_name="s"),
    compiler_params=pltpu.CompilerParams(needs_layout_passes=False),
    out_type=jax.ShapeDtypeStruct((2 * NP, HD), jnp.float32),
    scratch_types=(
        [pltpu.VMEM_SHARED((NP, HD), jnp.float32)]
        + [pltpu.VMEM((2 * CHUNK,), jnp.int32) for _ in range(NBUF)]
        + [pltpu.VMEM((CHUNK, HD), jnp.float32) for _ in range(NBUF)]
        + [pltpu.VMEM((CHUNK,), jnp.int32) for _ in range(DRING)]
        + [pltpu.SemaphoreType.DMA for _ in range(2 * NBUF + NBUF + DRING)]
    ),
)(_sc_agg_body)


def _gates(z):
    i = jax.nn.sigmoid(z[:, 0 * 256:1 * 256])
    g = jnp.tanh(z[:, 2 * 256:3 * 256])
    o = jax.nn.sigmoid(z[:, 3 * 256:4 * 256])
    cc = i * g
    hh = o * jnp.tanh(cc)
    return hh, cc


def _tc0_body(m_ref, w_ref, b_ref, h1s_ref, h1n_ref, c1n_ref):
    z = (jnp.dot(m_ref[0], w_ref[0], preferred_element_type=jnp.float32)
         + jnp.dot(m_ref[1], w_ref[1], preferred_element_type=jnp.float32)
         + b_ref[0:1, :])
    h1, c1 = _gates(z)
    h1s_ref[0] = h1[:, :HD]
    h1s_ref[1] = h1[:, HD:]
    h1n_ref[0] = h1
    c1n_ref[0] = c1


def _ln(x, g, b):
    m = jnp.mean(x, axis=-1, keepdims=True)
    v = jnp.mean((x - m) ** 2, axis=-1, keepdims=True)
    return (x - m) * jax.lax.rsqrt(v + 1e-5) * g + b


def _tc1_body(m_ref, w_ref, b_ref, g_ref, be_ref, ha_ref, ca_ref,
              h2_ref, c2_ref):
    z = (jnp.dot(m_ref[0], w_ref[0], preferred_element_type=jnp.float32)
         + jnp.dot(m_ref[1], w_ref[1], preferred_element_type=jnp.float32)
         + b_ref[0:1, :])
    h2, c2 = _gates(z)
    h2_ref[0] = _ln(h2, g_ref[0:1, :], be_ref[0:1, :])
    c2_ref[0] = _ln(c2, g_ref[0:1, :], be_ref[0:1, :])


def _tc_cell0(m_split, w_split, b8):
    return pl.pallas_call(
        _tc0_body,
        grid=(N // BLK,),
        in_specs=[
            pl.BlockSpec((2, BLK, HD), lambda i: (0, i, 0)),
            pl.BlockSpec((2, HD, G4), lambda i: (0, 0, 0)),
            pl.BlockSpec((8, G4), lambda i: (0, 0)),
        ],
        out_specs=[
            pl.BlockSpec((2, BLK, HD), lambda i: (0, i, 0)),
            pl.BlockSpec((1, BLK, D), lambda i: (0, i, 0)),
            pl.BlockSpec((1, BLK, D), lambda i: (0, i, 0)),
        ],
        out_shape=[
            jax.ShapeDtypeStruct((2, NP, HD), jnp.float32),
            jax.ShapeDtypeStruct((2, N, D), jnp.float32),
            jax.ShapeDtypeStruct((2, N, D), jnp.float32),
        ],
    )(m_split, w_split, b8)


def _tc_cell1(m_split, w_split, b8, g8, be8, h_all, c_all):
    return pl.pallas_call(
        _tc1_body,
        grid=(N // BLK,),
        in_specs=[
            pl.BlockSpec((2, BLK, HD), lambda i: (0, i, 0)),
            pl.BlockSpec((2, HD, G4), lambda i: (0, 0, 0)),
            pl.BlockSpec((8, G4), lambda i: (0, 0)),
            pl.BlockSpec((8, D), lambda i: (0, 0)),
            pl.BlockSpec((8, D), lambda i: (0, 0)),
            pl.BlockSpec(memory_space=pl.ANY),
            pl.BlockSpec(memory_space=pl.ANY),
        ],
        out_specs=[
            pl.BlockSpec((1, BLK, D), lambda i: (1, i, 0)),
            pl.BlockSpec((1, BLK, D), lambda i: (1, i, 0)),
        ],
        out_shape=[
            jax.ShapeDtypeStruct((2, N, D), jnp.float32),
            jax.ShapeDtypeStruct((2, N, D), jnp.float32),
        ],
        input_output_aliases={5: 0, 6: 1},
    )(m_split, w_split, b8, g8, be8, h_all, c_all)


def kernel(X, edge_index, edge_weight, Wx0, Wh0, b0, Wx1, Wh1, b1, ln_gamma, ln_beta):
    src = edge_index[0]
    dst = edge_index[1]
    pad = E_PAD - E
    pad_idx = jnp.arange(pad, dtype=jnp.int32) % N
    srcp = jnp.concatenate([src, pad_idx])
    dstp = jnp.concatenate([dst, pad_idx]).reshape(NS, NCHUNKS, CHUNK)
    wp = jnp.concatenate([edge_weight, jnp.zeros((pad,), jnp.float32)])
    wbits = jax.lax.bitcast_convert_type(wp, jnp.int32).reshape(NS, NCHUNKS, CHUNK)
    src2 = jnp.stack([srcp, srcp + NP]).reshape(2, NS, NCHUNKS, CHUNK)
    auxp = jnp.concatenate(
        [src2, jnp.broadcast_to(wbits[None], (2, NS, NCHUNKS, CHUNK))], axis=-1)
    Xp = jnp.pad(X, ((0, NP - N), (0, 0)))
    x_cat = jnp.concatenate([Xp[:, :HD], Xp[:, HD:]], axis=0)

    b0_8 = jnp.tile(b0[None, :], (8, 1))
    b1_8 = jnp.tile(b1[None, :], (8, 1))
    g8 = jnp.tile(ln_gamma[None, :], (8, 1))
    be8 = jnp.tile(ln_beta[None, :], (8, 1))
    w0s = Wx0.reshape(2, HD, G4)
    w1s = Wx1.reshape(2, HD, G4)

    m0 = _sc_agg(x_cat, auxp, dstp)
    h1s, h_all, c_all = _tc_cell0(m0.reshape(2, NP, HD), w0s, b0_8)
    m1 = _sc_agg(h1s.reshape(2 * NP, HD), auxp, dstp)
    hidden, cell = _tc_cell1(m1.reshape(2, NP, HD), w1s, b1_8, g8, be8,
                             h_all, c_all)
    return hidden, cell


# speedup vs baseline: 1.3140x; 1.0110x over previous
import functools

import jax
import jax.numpy as jnp
from jax import lax
from jax.experimental import pallas as pl
from jax.experimental.pallas import tpu as pltpu
from jax.experimental.pallas import tpu_sc as plsc

N = 10000
NP = 10240
D = 256
HD = 128
G4 = 4 * 256
E = 160000

NC = 2
NS = 16
LANES = 16

CHUNK = 80
EPT = 10240
NCHUNKS = EPT // CHUNK
E_PAD = NS * EPT
ROWS_PER_TILE = NP // NS

BLK = 2000


NBUF = 4
DRING = 8


def _sc_agg_body(x_hbm, aux_hbm, didx_hbm, out_hbm, acc_sh,
                 a0, a1, a2, a3, rb0, rb1, rb2, rb3,
                 d0, d1, d2, d3, d4, d5, d6, d7,
                 ga0, ga1, ga2, ga3, gg0, gg1, gg2, gg3,
                 gs0, gs1, gs2, gs3, gd0, gd1, gd2, gd3, gd5, gd6, gd7, gd8):
    c = lax.axis_index("c")
    sid = lax.axis_index("s")
    row0 = sid * ROWS_PER_TILE
    pltpu.sync_copy(x_hbm.at[pl.ds(c * NP + row0, ROWS_PER_TILE)],
                    acc_sh.at[pl.ds(row0, ROWS_PER_TILE)])
    plsc.subcore_barrier()

    aux = [a0, a1, a2, a3]
    rows = [rb0, rb1, rb2, rb3]
    didx = [d0, d1, d2, d3, d4, d5, d6, d7]
    asem = [ga0, ga1, ga2, ga3]
    gsem = [gg0, gg1, gg2, gg3]
    ssem = [gs0, gs1, gs2, gs3]
    dsem = [gd0, gd1, gd2, gd3, gd5, gd6, gd7, gd8]

    def issue_aux(ch, b):
        pltpu.async_copy(aux_hbm.at[c, sid, ch], aux[b], asem[b])

    def wait_aux(b):
        pltpu.make_async_copy(aux_hbm.at[c, sid, 0], aux[b], asem[b]).wait()

    def issue_didx(ch, b):
        pltpu.async_copy(didx_hbm.at[sid, ch], didx[b], dsem[b])

    def wait_didx(b):
        pltpu.make_async_copy(didx_hbm.at[sid, 0], didx[b], dsem[b]).wait()

    def issue_gather(b):
        pltpu.async_copy(x_hbm.at[aux[b].at[pl.ds(0, CHUNK)]], rows[b], gsem[b])

    def wait_gather(b):
        pltpu.make_async_copy(x_hbm.at[aux[0].at[pl.ds(0, CHUNK)]], rows[b],
                              gsem[b]).wait()

    def issue_scatter(b, db):
        pltpu.async_copy(rows[b], acc_sh.at[didx[db]], ssem[b], add=True)

    def wait_scatter(b):
        pltpu.make_async_copy(rows[b], acc_sh.at[didx[0]], ssem[b]).wait()

    def scale(b):
        def grp(g, carry):
            wg = plsc.bitcast(aux[b][pl.ds(CHUNK + g * LANES, LANES)],
                              jnp.float32)
            for el in range(LANES):
                spl = jnp.take_along_axis(wg, jnp.full((LANES,), el, jnp.int32),
                                          axis=0)
                e = g * LANES + el
                for j in range(HD // LANES):
                    sl = pl.ds(j * LANES, LANES)
                    rows[b][e, sl] = rows[b][e, sl] * spl
            return carry
        lax.fori_loop(0, CHUNK // LANES, grp, 0, unroll=False)

    for j in range(NBUF):
        issue_aux(j, j)
    for j in range(6):
        issue_didx(j, j)
    for j in range(2):
        wait_aux(j)
        issue_gather(j)

    def outer(k, carry):
        for b8 in range(DRING):
            ch = k * DRING + b8
            b = b8 % NBUF
            wait_gather(b)
            scale(b)
            wait_didx(b8)
            issue_scatter(b, b8)
            nb = (b + 2) % NBUF

            @pl.when(ch + 2 < NCHUNKS)
            def _():
                @pl.when(ch >= 2)
                def _():
                    wait_scatter(nb)
                wait_aux(nb)
                issue_gather(nb)

            @pl.when(ch + NBUF < NCHUNKS)
            def _():
                issue_aux(ch + NBUF, b)

            @pl.when(ch + 6 < NCHUNKS)
            def _():
                issue_didx(ch + 6, (b8 + 6) % DRING)
        return carry

    lax.fori_loop(0, NCHUNKS // DRING, outer, 0, unroll=False)
    for j in range(NBUF):
        wait_scatter(j)
    plsc.subcore_barrier()
    pltpu.sync_copy(acc_sh.at[pl.ds(row0, ROWS_PER_TILE)],
                    out_hbm.at[pl.ds(c * NP + row0, ROWS_PER_TILE)])


_sc_agg = functools.partial(
    pl.kernel,
    mesh=plsc.VectorSubcoreMesh(core_axis_name="c", subcore_axis_name="s"),
    compiler_params=pltpu.CompilerParams(needs_layout_passes=False),
    out_type=jax.ShapeDtypeStruct((2 * NP, HD), jnp.float32),
    scratch_types=(
        [pltpu.VMEM_SHARED((NP, HD), jnp.float32)]
        + [pltpu.VMEM((2 * CHUNK,), jnp.int32) for _ in range(NBUF)]
        + [pltpu.VMEM((CHUNK, HD), jnp.float32) for _ in range(NBUF)]
        + [pltpu.VMEM((CHUNK,), jnp.int32) for _ in range(DRING)]
        + [pltpu.SemaphoreType.DMA for _ in range(2 * NBUF + NBUF + DRING)]
    ),
)(_sc_agg_body)


def _gates(z):
    i = jax.nn.sigmoid(z[:, 0 * 256:1 * 256])
    g = jnp.tanh(z[:, 2 * 256:3 * 256])
    o = jax.nn.sigmoid(z[:, 3 * 256:4 * 256])
    cc = i * g
    hh = o * jnp.tanh(cc)
    return hh, cc


def _tc0_body(m_ref, w_ref, b_ref, h1s_ref, h1n_ref, c1n_ref):
    z = (jnp.dot(m_ref[0], w_ref[0], preferred_element_type=jnp.float32)
         + jnp.dot(m_ref[1], w_ref[1], preferred_element_type=jnp.float32)
         + b_ref[0:1, :])
    h1, c1 = _gates(z)
    h1s_ref[0] = h1[:, :HD]
    h1s_ref[1] = h1[:, HD:]
    h1n_ref[0] = h1
    c1n_ref[0] = c1


def _ln(x, g, b):
    m = jnp.mean(x, axis=-1, keepdims=True)
    v = jnp.mean((x - m) ** 2, axis=-1, keepdims=True)
    return (x - m) * jax.lax.rsqrt(v + 1e-5) * g + b


def _tc1_body(m_ref, w_ref, b_ref, g_ref, be_ref, ha_ref, ca_ref,
              h2_ref, c2_ref):
    z = (jnp.dot(m_ref[0], w_ref[0], preferred_element_type=jnp.float32)
         + jnp.dot(m_ref[1], w_ref[1], preferred_element_type=jnp.float32)
         + b_ref[0:1, :])
    h2, c2 = _gates(z)
    h2_ref[0] = _ln(h2, g_ref[0:1, :], be_ref[0:1, :])
    c2_ref[0] = _ln(c2, g_ref[0:1, :], be_ref[0:1, :])


def _tc_cell0(m_split, w_split, b8):
    return pl.pallas_call(
        _tc0_body,
        grid=(N // BLK,),
        in_specs=[
            pl.BlockSpec((2, BLK, HD), lambda i: (0, i, 0)),
            pl.BlockSpec((2, HD, G4), lambda i: (0, 0, 0)),
            pl.BlockSpec((8, G4), lambda i: (0, 0)),
        ],
        out_specs=[
            pl.BlockSpec((2, BLK, HD), lambda i: (0, i, 0)),
            pl.BlockSpec((1, BLK, D), lambda i: (0, i, 0)),
            pl.BlockSpec((1, BLK, D), lambda i: (0, i, 0)),
        ],
        out_shape=[
            jax.ShapeDtypeStruct((2, NP, HD), jnp.float32),
            jax.ShapeDtypeStruct((2, N, D), jnp.float32),
            jax.ShapeDtypeStruct((2, N, D), jnp.float32),
        ],
    )(m_split, w_split, b8)


def _tc_cell1(m_split, w_split, b8, g8, be8, h_all, c_all):
    return pl.pallas_call(
        _tc1_body,
        grid=(N // BLK,),
        in_specs=[
            pl.BlockSpec((2, BLK, HD), lambda i: (0, i, 0)),
            pl.BlockSpec((2, HD, G4), lambda i: (0, 0, 0)),
            pl.BlockSpec((8, G4), lambda i: (0, 0)),
            pl.BlockSpec((8, D), lambda i: (0, 0)),
            pl.BlockSpec((8, D), lambda i: (0, 0)),
            pl.BlockSpec(memory_space=pl.ANY),
            pl.BlockSpec(memory_space=pl.ANY),
        ],
        out_specs=[
            pl.BlockSpec((1, BLK, D), lambda i: (1, i, 0)),
            pl.BlockSpec((1, BLK, D), lambda i: (1, i, 0)),
        ],
        out_shape=[
            jax.ShapeDtypeStruct((2, N, D), jnp.float32),
            jax.ShapeDtypeStruct((2, N, D), jnp.float32),
        ],
        input_output_aliases={5: 0, 6: 1},
    )(m_split, w_split, b8, g8, be8, h_all, c_all)


def kernel(X, edge_index, edge_weight, Wx0, Wh0, b0, Wx1, Wh1, b1, ln_gamma, ln_beta):
    src = edge_index[0]
    dst = edge_index[1]
    pad = E_PAD - E
    pad_idx = jnp.arange(pad, dtype=jnp.int32) % N
    srcp = jnp.concatenate([src, pad_idx])
    dstp = jnp.concatenate([dst, pad_idx]).reshape(NS, NCHUNKS, CHUNK)
    wp = jnp.concatenate([edge_weight, jnp.zeros((pad,), jnp.float32)])
    wbits = jax.lax.bitcast_convert_type(wp, jnp.int32).reshape(NS, NCHUNKS, CHUNK)
    src2 = jnp.stack([srcp, srcp + NP]).reshape(2, NS, NCHUNKS, CHUNK)
    auxp = jnp.concatenate(
        [src2, jnp.broadcast_to(wbits[None], (2, NS, NCHUNKS, CHUNK))], axis=-1)
    Xp = jnp.pad(X, ((0, NP - N), (0, 0)))
    x_cat = jnp.concatenate([Xp[:, :HD], Xp[:, HD:]], axis=0)

    b0_8 = jnp.tile(b0[None, :], (8, 1))
    b1_8 = jnp.tile(b1[None, :], (8, 1))
    g8 = jnp.tile(ln_gamma[None, :], (8, 1))
    be8 = jnp.tile(ln_beta[None, :], (8, 1))
    w0s = Wx0.reshape(2, HD, G4)
    w1s = Wx1.reshape(2, HD, G4)

    m0 = _sc_agg(x_cat, auxp, dstp)
    h1s, h_all, c_all = _tc_cell0(m0.reshape(2, NP, HD), w0s, b0_8)
    m1 = _sc_agg(h1s.reshape(2 * NP, HD), auxp, dstp)
    hidden, cell = _tc_cell1(m1.reshape(2, NP, HD), w1s, b1_8, g8, be8,
                             h_all, c_all)
    return hidden, cell
